# baseline (speedup 1.0000x reference)
import functools

import jax
import jax.numpy as jnp
from jax import lax
from jax.experimental import pallas as pl
from jax.experimental.pallas import tpu as pltpu
from jax.experimental.pallas import tpu_sc as plsc

N = 10000
E = 320000
D = 128

NC = 2
NS = 16
NW = NC * NS
C = 64
GC = 8
NG0 = 28
NG1 = 12
NGMAX = max(NG0, NG1)
GROUP_E = GC * C
EPAD = NS * (NG0 + NG1) * GROUP_E
NBUF = 4
LA = 2
NPAD = 10240
RPT = NPAD // NS
BN = 256
GRID = NPAD // BN

_mesh = plsc.VectorSubcoreMesh(core_axis_name="c", subcore_axis_name="s")
_sc_params = pltpu.CompilerParams(needs_layout_passes=False)


@functools.partial(
    pl.kernel,
    out_type=jax.ShapeDtypeStruct((NC, NPAD), jnp.float32),
    mesh=_mesh,
    scratch_types=[
        pltpu.VMEM((NGMAX, GC, C), jnp.int32),
        pltpu.VMEM((NGMAX, GROUP_E), jnp.float32),
        pltpu.VMEM((RPT,), jnp.float32),
        pltpu.VMEM_SHARED((NPAD,), jnp.float32),
        pltpu.SemaphoreType.DMA,
    ],
    compiler_params=_sc_params,
)
def _deg_kernel(col_hbm, ew_hbm, degp_hbm, col_v, ew_v, zbuf, acc, dsem):
    cid = lax.axis_index("c")
    sid = lax.axis_index("s")
    wid = cid * NS + sid
    ng = lax.select(cid == 0, jnp.int32(NG0), jnp.int32(NG1))

    pltpu.sync_copy(col_hbm.at[wid], col_v)
    pltpu.sync_copy(ew_hbm.at[wid], ew_v)

    zeros = jnp.zeros((16,), jnp.float32)

    def zb(i, _):
        zbuf[pl.ds(i * 16, 16)] = zeros
        return 0

    lax.fori_loop(0, RPT // 16, zb, 0)
    pltpu.sync_copy(zbuf, acc.at[pl.ds(sid * RPT, RPT)])
    plsc.subcore_barrier()

    def fire(g, _):
        def fire_q(q, _):
            pltpu.async_copy(ew_v.at[g, pl.ds(q * C, C)],
                             acc.at[col_v.at[g, q]], dsem, add=True)
            return 0

        lax.fori_loop(0, GC, fire_q, 0)
        return 0

    lax.fori_loop(0, ng, fire, 0)

    def drain(j, _):
        pltpu.make_async_copy(ew_v.at[0, pl.ds(0, C)],
                              acc.at[col_v.at[0, 0]], dsem).wait()
        return 0

    lax.fori_loop(0, ng * GC, drain, 0)
    plsc.subcore_barrier()

    pltpu.sync_copy(acc.at[pl.ds(sid * RPT, RPT)],
                    degp_hbm.at[cid, pl.ds(sid * RPT, RPT)])


@functools.partial(
    pl.kernel,
    out_type=jax.ShapeDtypeStruct((NC, NPAD, D), jnp.float32),
    mesh=_mesh,
    scratch_types=(
        [
            pltpu.VMEM((2, GC, C), jnp.int32),
            pltpu.VMEM((2, GC, C), jnp.int32),
            pltpu.VMEM((2 * GROUP_E,), jnp.float32),
            pltpu.VMEM((NBUF, C, D), jnp.float32),
            pltpu.VMEM_SHARED((NPAD, D), jnp.float32),
            pltpu.SemaphoreType.DMA,
        ]
        + [pltpu.SemaphoreType.DMA] * NBUF
        + [pltpu.SemaphoreType.DMA] * NBUF
    ),
    compiler_params=_sc_params,
)
def _agg_kernel(hp_hbm, row_hbm, col_hbm, ew_hbm, part_hbm,
                row_g, col_g, ew_g, gbuf, acc, egsem, *sems):
    gsem = sems[:NBUF]
    ssem = sems[NBUF:]
    cid = lax.axis_index("c")
    sid = lax.axis_index("s")
    wid = cid * NS + sid
    ng = lax.select(cid == 0, jnp.int32(NG0), jnp.int32(NG1))

    zeros = jnp.zeros((16,), jnp.float32)

    def zrow(r, _):
        for k in range(D // 16):
            gbuf[0, r, pl.ds(k * 16, 16)] = zeros
        return 0

    lax.fori_loop(0, C, zrow, 0)
    for u in range(RPT // C):
        pltpu.sync_copy(gbuf.at[0], acc.at[pl.ds(sid * RPT + u * C, C)])
    plsc.subcore_barrier()

    def stage(g, p):
        pltpu.async_copy(row_hbm.at[wid, g], row_g.at[p], egsem)
        pltpu.async_copy(col_hbm.at[wid, g], col_g.at[p], egsem)
        pltpu.async_copy(ew_hbm.at[wid, g],
                         ew_g.at[pl.ds(p * GROUP_E, GROUP_E)], egsem)

    def stage_wait(g, p):
        pltpu.make_async_copy(row_hbm.at[wid, g], row_g.at[p], egsem).wait()
        pltpu.make_async_copy(col_hbm.at[wid, g], col_g.at[p], egsem).wait()
        pltpu.make_async_copy(ew_hbm.at[wid, g],
                              ew_g.at[pl.ds(p * GROUP_E, GROUP_E)],
                              egsem).wait()

    def gather_start(cig, p, b):
        pltpu.async_copy(hp_hbm.at[row_g.at[p, cig]], gbuf.at[b], gsem[b])

    def gather_wait(cig, p, b):
        pltpu.make_async_copy(hp_hbm.at[row_g.at[p, cig]], gbuf.at[b],
                              gsem[b]).wait()

    def scat_start(cig, p, b):
        pltpu.async_copy(gbuf.at[b], acc.at[col_g.at[p, cig]], ssem[b],
                         add=True)

    def scat_wait(b):
        pltpu.make_async_copy(gbuf.at[0], acc.at[col_g.at[0, 0]],
                              ssem[b]).wait()

    def scale(cig, p, b):
        base = (p * GC + cig) * C

        def ebody(e, _):
            ewb = plsc.load_gather(
                ew_g, [jnp.full((16,), base + e, jnp.int32)])
            for k in range(D // 16):
                sl = pl.ds(k * 16, 16)
                gbuf[b, e, sl] = gbuf[b, e, sl] * ewb
            return 0

        lax.fori_loop(0, C, ebody, 0, unroll=4)

    def group_body(g, p):
        stage_wait(g, p)

        @pl.when(g + 1 < ng)
        def _():
            stage(g + 1, 1 - p)

        for j in range(LA):
            gather_start(j, p, j)

        def round_body(r, _):
            for b in range(NBUF):
                cig = r * NBUF + b
                gather_wait(cig, p, b)
                scale(cig, p, b)
                scat_start(cig, p, b)
                nxt = cig + LA
                b2 = (b + LA) % NBUF

                @pl.when(nxt < GC)
                def _():
                    @pl.when(nxt >= NBUF)
                    def _():
                        scat_wait(b2)

                    gather_start(nxt, p, b2)

            return 0

        lax.fori_loop(0, GC // NBUF, round_body, 0)
        for b in range(NBUF):
            scat_wait(b)

    @pl.when(ng > 0)
    def _():
        stage(0, 0)

    def outer(gg, _):
        group_body(gg * 2, 0)
        group_body(gg * 2 + 1, 1)
        return 0

    lax.fori_loop(0, ng // 2, outer, 0)
    plsc.subcore_barrier()

    pltpu.sync_copy(acc.at[pl.ds(sid * RPT, RPT)],
                    part_hbm.at[cid, pl.ds(sid * RPT, RPT)])


def _mm_scale_body(deg_ref, x_ref, w_ref, hp_ref, dis_ref):
    deg = jnp.sum(deg_ref[...], axis=0) + 1.0
    dis = jnp.where(deg > 0, lax.rsqrt(deg), 0.0)
    dis_ref[...] = dis
    h = lax.dot_general(x_ref[...], w_ref[...], (((1,), (1,)), ((), ())),
                        preferred_element_type=jnp.float32)
    hp_ref[...] = h * dis[:, None]


def _mid_body(part_ref, hp_ref, dis_ref, b_ref, w_ref, out_ref):
    aggr = part_ref[0] + part_ref[1] + hp_ref[...]
    dis = dis_ref[...]
    h1 = jnp.maximum(aggr * dis[:, None] + b_ref[...][None, :], 0.0)
    h2 = lax.dot_general(h1, w_ref[...], (((1,), (1,)), ((), ())),
                         preferred_element_type=jnp.float32)
    out_ref[...] = h2 * dis[:, None]


def _final_body(part_ref, hp_ref, dis_ref, b_ref, out_ref):
    aggr = part_ref[0] + part_ref[1] + hp_ref[...]
    out_ref[...] = jnp.maximum(
        aggr * dis_ref[...][:, None] + b_ref[...][None, :], 0.0)


def _split_edges(flat, shape_tail):
    n0 = NS * NG0 * GROUP_E
    a0 = flat[:n0].reshape(NS, NG0, *shape_tail)
    a1 = flat[n0:].reshape(NS, NG1, *shape_tail)
    padw = [(0, 0), (0, NGMAX - NG0)] + [(0, 0)] * len(shape_tail)
    a0 = jnp.pad(a0, padw)
    padw = [(0, 0), (0, NGMAX - NG1)] + [(0, 0)] * len(shape_tail)
    a1 = jnp.pad(a1, padw)
    return jnp.concatenate([a0, a1], axis=0)


def kernel(x, edge_index, edge_attr, W1, b1, W2, b2):
    row = edge_index[0]
    col = edge_index[1]
    pad = EPAD - E
    rowp = jnp.concatenate([row, jnp.zeros((pad,), row.dtype)])
    colp = jnp.concatenate([col, jnp.zeros((pad,), col.dtype)])
    ewp = jnp.concatenate([edge_attr, jnp.zeros((pad,), edge_attr.dtype)])
    rowp = _split_edges(rowp, (GC, C))
    colp = _split_edges(colp, (GC, C))
    ewp = _split_edges(ewp, (GROUP_E,))
    xp = jnp.pad(x[0], ((0, NPAD - N), (0, 0)))

    degp = _deg_kernel(colp, ewp)

    hp1, dis = pl.pallas_call(
        _mm_scale_body,
        grid=(GRID,),
        in_specs=[
            pl.BlockSpec((NC, BN), lambda i: (0, i)),
            pl.BlockSpec((BN, D), lambda i: (i, 0)),
            pl.BlockSpec((D, D), lambda i: (0, 0)),
        ],
        out_specs=[
            pl.BlockSpec((BN, D), lambda i: (i, 0)),
            pl.BlockSpec((BN,), lambda i: (i,)),
        ],
        out_shape=[
            jax.ShapeDtypeStruct((NPAD, D), jnp.float32),
            jax.ShapeDtypeStruct((NPAD,), jnp.float32),
        ],
    )(degp, xp, W1)

    part1 = _agg_kernel(hp1, rowp, colp, ewp)

    hp2 = pl.pallas_call(
        _mid_body,
        grid=(GRID,),
        in_specs=[
            pl.BlockSpec((NC, BN, D), lambda i: (0, i, 0)),
            pl.BlockSpec((BN, D), lambda i: (i, 0)),
            pl.BlockSpec((BN,), lambda i: (i,)),
            pl.BlockSpec((D,), lambda i: (0,)),
            pl.BlockSpec((D, D), lambda i: (0, 0)),
        ],
        out_specs=pl.BlockSpec((BN, D), lambda i: (i, 0)),
        out_shape=jax.ShapeDtypeStruct((NPAD, D), jnp.float32),
    )(part1, hp1, dis, b1, W2)

    part2 = _agg_kernel(hp2, rowp, colp, ewp)

    y = pl.pallas_call(
        _final_body,
        grid=(GRID,),
        in_specs=[
            pl.BlockSpec((NC, BN, D), lambda i: (0, i, 0)),
            pl.BlockSpec((BN, D), lambda i: (i, 0)),
            pl.BlockSpec((BN,), lambda i: (i,)),
            pl.BlockSpec((D,), lambda i: (0,)),
        ],
        out_specs=pl.BlockSpec((BN, D), lambda i: (i, 0)),
        out_shape=jax.ShapeDtypeStruct((NPAD, D), jnp.float32),
    )(part2, hp2, dis, b2)

    return y[:N][None]


# speedup vs baseline: 1.1515x; 1.1515x over previous
import functools

import jax
import jax.numpy as jnp
from jax import lax
from jax.experimental import pallas as pl
from jax.experimental.pallas import tpu as pltpu
from jax.experimental.pallas import tpu_sc as plsc

N = 10000
E = 320000
D = 128

NC = 2
NS = 16
NW = NC * NS
C = 64
GC = 8
NG0 = 30
NG1 = 10
NGMAX = max(NG0, NG1)
GROUP_E = GC * C
EPAD = NS * (NG0 + NG1) * GROUP_E
NBUF = 4
LA = 2
NPAD = 10240
RPT = NPAD // NS
BN = 256
GRID = NPAD // BN

_mesh = plsc.VectorSubcoreMesh(core_axis_name="c", subcore_axis_name="s")
_sc_params = pltpu.CompilerParams(needs_layout_passes=False)


@functools.partial(
    pl.kernel,
    out_type=jax.ShapeDtypeStruct((NC, NPAD), jnp.float32),
    mesh=_mesh,
    scratch_types=[
        pltpu.VMEM((NGMAX, GC, C), jnp.int32),
        pltpu.VMEM((NGMAX, GROUP_E), jnp.float32),
        pltpu.VMEM((RPT,), jnp.float32),
        pltpu.VMEM_SHARED((NPAD,), jnp.float32),
        pltpu.SemaphoreType.DMA,
    ],
    compiler_params=_sc_params,
)
def _deg_kernel(col_hbm, ew_hbm, degp_hbm, col_v, ew_v, zbuf, acc, dsem):
    cid = lax.axis_index("c")
    sid = lax.axis_index("s")
    wid = cid * NS + sid
    ng = lax.select(cid == 0, jnp.int32(NG0), jnp.int32(NG1))

    pltpu.sync_copy(col_hbm.at[wid], col_v)
    pltpu.sync_copy(ew_hbm.at[wid], ew_v)

    zeros = jnp.zeros((16,), jnp.float32)

    def zb(i, _):
        zbuf[pl.ds(i * 16, 16)] = zeros
        return 0

    lax.fori_loop(0, RPT // 16, zb, 0)
    pltpu.sync_copy(zbuf, acc.at[pl.ds(sid * RPT, RPT)])
    plsc.subcore_barrier()

    def fire(g, _):
        def fire_q(q, _):
            pltpu.async_copy(ew_v.at[g, pl.ds(q * C, C)],
                             acc.at[col_v.at[g, q]], dsem, add=True)
            return 0

        lax.fori_loop(0, GC, fire_q, 0)
        return 0

    lax.fori_loop(0, ng, fire, 0)

    def drain(j, _):
        pltpu.make_async_copy(ew_v.at[0, pl.ds(0, C)],
                              acc.at[col_v.at[0, 0]], dsem).wait()
        return 0

    lax.fori_loop(0, ng * GC, drain, 0)
    plsc.subcore_barrier()

    pltpu.sync_copy(acc.at[pl.ds(sid * RPT, RPT)],
                    degp_hbm.at[cid, pl.ds(sid * RPT, RPT)])


@functools.partial(
    pl.kernel,
    out_type=jax.ShapeDtypeStruct((NC, NPAD, D), jnp.float32),
    mesh=_mesh,
    scratch_types=(
        [
            pltpu.VMEM((2, GC, C), jnp.int32),
            pltpu.VMEM((2, GC, C), jnp.int32),
            pltpu.VMEM((2 * GROUP_E,), jnp.float32),
            pltpu.VMEM((NBUF, C, D), jnp.float32),
            pltpu.VMEM_SHARED((NPAD, D), jnp.float32),
            pltpu.SemaphoreType.DMA,
        ]
        + [pltpu.SemaphoreType.DMA] * NBUF
        + [pltpu.SemaphoreType.DMA] * NBUF
        + [pltpu.SemaphoreType.DMA] * NBUF
    ),
    compiler_params=_sc_params,
)
def _agg_kernel(hp_hbm, row_hbm, col_hbm, ew_hbm, part_hbm,
                row_g, col_g, ew_g, gbuf, acc, egsem, *sems):
    gsem = sems[:NBUF]
    gsem2 = sems[NBUF:2 * NBUF]
    ssem = sems[2 * NBUF:]
    cid = lax.axis_index("c")
    sid = lax.axis_index("s")
    wid = cid * NS + sid
    ng = lax.select(cid == 0, jnp.int32(NG0), jnp.int32(NG1))

    zeros = jnp.zeros((16,), jnp.float32)

    def zrow(r, _):
        for k in range(D // 16):
            gbuf[0, r, pl.ds(k * 16, 16)] = zeros
        return 0

    lax.fori_loop(0, C, zrow, 0)
    for u in range(RPT // C):
        pltpu.sync_copy(gbuf.at[0], acc.at[pl.ds(sid * RPT + u * C, C)])
    plsc.subcore_barrier()

    def stage(g, p):
        pltpu.async_copy(row_hbm.at[wid, g], row_g.at[p], egsem)
        pltpu.async_copy(col_hbm.at[wid, g], col_g.at[p], egsem)
        pltpu.async_copy(ew_hbm.at[wid, g],
                         ew_g.at[pl.ds(p * GROUP_E, GROUP_E)], egsem)

    def stage_wait(g, p):
        pltpu.make_async_copy(row_hbm.at[wid, g], row_g.at[p], egsem).wait()
        pltpu.make_async_copy(col_hbm.at[wid, g], col_g.at[p], egsem).wait()
        pltpu.make_async_copy(ew_hbm.at[wid, g],
                              ew_g.at[pl.ds(p * GROUP_E, GROUP_E)],
                              egsem).wait()

    H = C // 2

    def gather_start(cig, p, b):
        pltpu.async_copy(hp_hbm.at[row_g.at[p, cig, pl.ds(0, H)]],
                         gbuf.at[b, pl.ds(0, H)], gsem[b])
        pltpu.async_copy(hp_hbm.at[row_g.at[p, cig, pl.ds(H, H)]],
                         gbuf.at[b, pl.ds(H, H)], gsem2[b])

    def gather_wait(cig, p, b):
        pltpu.make_async_copy(hp_hbm.at[row_g.at[p, cig, pl.ds(0, H)]],
                              gbuf.at[b, pl.ds(0, H)], gsem[b]).wait()
        pltpu.make_async_copy(hp_hbm.at[row_g.at[p, cig, pl.ds(H, H)]],
                              gbuf.at[b, pl.ds(H, H)], gsem2[b]).wait()

    def scat_start(cig, p, b):
        pltpu.async_copy(gbuf.at[b], acc.at[col_g.at[p, cig]], ssem[b],
                         add=True)

    def scat_wait(b):
        pltpu.make_async_copy(gbuf.at[0], acc.at[col_g.at[0, 0]],
                              ssem[b]).wait()

    def scale(cig, p, b):
        base = (p * GC + cig) * C

        def ebody(e, _):
            ewb = plsc.load_gather(
                ew_g, [jnp.full((16,), base + e, jnp.int32)])
            for k in range(D // 16):
                sl = pl.ds(k * 16, 16)
                gbuf[b, e, sl] = gbuf[b, e, sl] * ewb
            return 0

        lax.fori_loop(0, C, ebody, 0, unroll=4)

    def group_body(g, p):
        stage_wait(g, p)

        @pl.when(g + 1 < ng)
        def _():
            stage(g + 1, 1 - p)

        for j in range(LA):
            gather_start(j, p, j)

        def round_body(r, _):
            for b in range(NBUF):
                cig = r * NBUF + b
                gather_wait(cig, p, b)
                scale(cig, p, b)
                scat_start(cig, p, b)
                nxt = cig + LA
                b2 = (b + LA) % NBUF

                @pl.when(nxt < GC)
                def _():
                    @pl.when(nxt >= NBUF)
                    def _():
                        scat_wait(b2)

                    gather_start(nxt, p, b2)

            return 0

        lax.fori_loop(0, GC // NBUF, round_body, 0)
        for b in range(NBUF):
            scat_wait(b)

    @pl.when(ng > 0)
    def _():
        stage(0, 0)

    def outer(gg, _):
        group_body(gg * 2, 0)
        group_body(gg * 2 + 1, 1)
        return 0

    lax.fori_loop(0, ng // 2, outer, 0)
    plsc.subcore_barrier()

    pltpu.sync_copy(acc.at[pl.ds(sid * RPT, RPT)],
                    part_hbm.at[cid, pl.ds(sid * RPT, RPT)])


def _mm_scale_body(deg_ref, x_ref, w_ref, hp_ref, dis_ref):
    deg = jnp.sum(deg_ref[...], axis=0) + 1.0
    dis = jnp.where(deg > 0, lax.rsqrt(deg), 0.0)
    dis_ref[...] = dis
    h = lax.dot_general(x_ref[...], w_ref[...], (((1,), (1,)), ((), ())),
                        preferred_element_type=jnp.float32)
    hp_ref[...] = h * dis[:, None]


def _mid_body(part_ref, hp_ref, dis_ref, b_ref, w_ref, out_ref):
    aggr = part_ref[0] + part_ref[1] + hp_ref[...]
    dis = dis_ref[...]
    h1 = jnp.maximum(aggr * dis[:, None] + b_ref[...][None, :], 0.0)
    h2 = lax.dot_general(h1, w_ref[...], (((1,), (1,)), ((), ())),
                         preferred_element_type=jnp.float32)
    out_ref[...] = h2 * dis[:, None]


def _final_body(part_ref, hp_ref, dis_ref, b_ref, out_ref):
    aggr = part_ref[0] + part_ref[1] + hp_ref[...]
    out_ref[...] = jnp.maximum(
        aggr * dis_ref[...][:, None] + b_ref[...][None, :], 0.0)


def _split_edges(flat, shape_tail):
    n0 = NS * NG0 * GROUP_E
    a0 = flat[:n0].reshape(NS, NG0, *shape_tail)
    a1 = flat[n0:].reshape(NS, NG1, *shape_tail)
    padw = [(0, 0), (0, NGMAX - NG0)] + [(0, 0)] * len(shape_tail)
    a0 = jnp.pad(a0, padw)
    padw = [(0, 0), (0, NGMAX - NG1)] + [(0, 0)] * len(shape_tail)
    a1 = jnp.pad(a1, padw)
    return jnp.concatenate([a0, a1], axis=0)


def kernel(x, edge_index, edge_attr, W1, b1, W2, b2):
    row = edge_index[0]
    col = edge_index[1]
    pad = EPAD - E
    rowp = jnp.concatenate([row, jnp.zeros((pad,), row.dtype)])
    colp = jnp.concatenate([col, jnp.zeros((pad,), col.dtype)])
    ewp = jnp.concatenate([edge_attr, jnp.zeros((pad,), edge_attr.dtype)])
    rowp = _split_edges(rowp, (GC, C))
    colp = _split_edges(colp, (GC, C))
    ewp = _split_edges(ewp, (GROUP_E,))
    xp = jnp.pad(x[0], ((0, NPAD - N), (0, 0)))

    degp = _deg_kernel(colp, ewp)

    hp1, dis = pl.pallas_call(
        _mm_scale_body,
        grid=(GRID,),
        in_specs=[
            pl.BlockSpec((NC, BN), lambda i: (0, i)),
            pl.BlockSpec((BN, D), lambda i: (i, 0)),
            pl.BlockSpec((D, D), lambda i: (0, 0)),
        ],
        out_specs=[
            pl.BlockSpec((BN, D), lambda i: (i, 0)),
            pl.BlockSpec((BN,), lambda i: (i,)),
        ],
        out_shape=[
            jax.ShapeDtypeStruct((NPAD, D), jnp.float32),
            jax.ShapeDtypeStruct((NPAD,), jnp.float32),
        ],
    )(degp, xp, W1)

    part1 = _agg_kernel(hp1, rowp, colp, ewp)

    hp2 = pl.pallas_call(
        _mid_body,
        grid=(GRID,),
        in_specs=[
            pl.BlockSpec((NC, BN, D), lambda i: (0, i, 0)),
            pl.BlockSpec((BN, D), lambda i: (i, 0)),
            pl.BlockSpec((BN,), lambda i: (i,)),
            pl.BlockSpec((D,), lambda i: (0,)),
            pl.BlockSpec((D, D), lambda i: (0, 0)),
        ],
        out_specs=pl.BlockSpec((BN, D), lambda i: (i, 0)),
        out_shape=jax.ShapeDtypeStruct((NPAD, D), jnp.float32),
    )(part1, hp1, dis, b1, W2)

    part2 = _agg_kernel(hp2, rowp, colp, ewp)

    y = pl.pallas_call(
        _final_body,
        grid=(GRID,),
        in_specs=[
            pl.BlockSpec((NC, BN, D), lambda i: (0, i, 0)),
            pl.BlockSpec((BN, D), lambda i: (i, 0)),
            pl.BlockSpec((BN,), lambda i: (i,)),
            pl.BlockSpec((D,), lambda i: (0,)),
        ],
        out_specs=pl.BlockSpec((BN, D), lambda i: (i, 0)),
        out_shape=jax.ShapeDtypeStruct((NPAD, D), jnp.float32),
    )(part2, hp2, dis, b2)

    return y[:N][None]
